# Initial kernel scaffold; baseline (speedup 1.0000x reference)
import jax
import jax.numpy as jnp
from jax.experimental import pallas as pl


def kernel(x, edge_index_l0, edge_index_l1, W_l0, b_l0, W_r0, b_r0, W_l1, b_l1, W_r1, b_r1):
    raise NotImplementedError("write your pallas kernel here")


# speedup vs baseline: 3.2946x; 3.2946x over previous
import functools

import jax
import jax.numpy as jnp
from jax import lax
from jax.experimental import pallas as pl
from jax.experimental.pallas import tpu as pltpu
from jax.experimental.pallas import tpu_sc as plsc

N = 10000
E = 320000
D = 128

NC = 2
NS = 16
DH = D // NC
EPS = E // NS
C = 80
NCHUNK = EPS // C
N_PAD = 10240
RPS = N_PAD // NS
F = 128
CW = 16

_f32 = jnp.float32


def _sc_body(ya_hbm, yb_hbm, src_hbm, dst_hbm, sa_out, sb_out, cnt_out,
             src_idx, dst_idx, rows, ones, zbuf, zcnt, acc_sh, cnt_sh, sem):
    cid = lax.axis_index("c")
    sid = lax.axis_index("s")

    def _zrow(i, _):
        for j in range(DH // 16):
            zbuf[i, pl.ds(j * 16, 16)] = jnp.zeros((16,), _f32)
        return 0
    lax.fori_loop(0, F, _zrow, 0)

    def _zcrow(i, _):
        zcnt[i, pl.ds(0, CW)] = jnp.zeros((CW,), _f32)
        return 0
    lax.fori_loop(0, RPS, _zcrow, 0)

    def _orow(i, _):
        ones[i, pl.ds(0, CW)] = jnp.ones((CW,), _f32)
        return 0
    lax.fori_loop(0, C, _orow, 0)

    for k in range(RPS // F):
        pltpu.sync_copy(zbuf, acc_sh.at[pl.ds(sid * RPS + k * F, F)])
    pltpu.sync_copy(zcnt, cnt_sh.at[pl.ds(sid * RPS, RPS)])
    plsc.subcore_barrier()

    ebase = sid * EPS

    def _edges(y_hbm, with_counts):
        def _chunk(i, _):
            b = pl.multiple_of(ebase + i * C, 8)
            pltpu.sync_copy(src_hbm.at[pl.ds(b, C)], src_idx)
            pltpu.sync_copy(dst_hbm.at[pl.ds(b, C)], dst_idx)
            pltpu.async_copy(y_hbm.at[src_idx], rows, sem).wait()
            pltpu.sync_copy(rows, acc_sh.at[dst_idx], add=True)
            if with_counts:
                pltpu.sync_copy(ones, cnt_sh.at[dst_idx], add=True)
            return 0
        lax.fori_loop(0, NCHUNK, _chunk, 0)

    @pl.when(cid == 0)
    def _():
        _edges(ya_hbm, True)

    @pl.when(cid == 1)
    def _():
        _edges(yb_hbm, False)

    plsc.subcore_barrier()

    @pl.when(cid == 0)
    def _():
        for k in range(RPS // F):
            r0 = sid * RPS + k * F
            pltpu.sync_copy(acc_sh.at[pl.ds(r0, F)], sa_out.at[pl.ds(r0, F)])
        pltpu.sync_copy(cnt_sh.at[pl.ds(sid * RPS, RPS)],
                        cnt_out.at[pl.ds(sid * RPS, RPS)])

    @pl.when(cid == 1)
    def _():
        for k in range(RPS // F):
            r0 = sid * RPS + k * F
            pltpu.sync_copy(acc_sh.at[pl.ds(r0, F)], sb_out.at[pl.ds(r0, F)])


_sc_agg = functools.partial(
    pl.kernel,
    out_type=(
        jax.ShapeDtypeStruct((N_PAD, DH), _f32),
        jax.ShapeDtypeStruct((N_PAD, DH), _f32),
        jax.ShapeDtypeStruct((N_PAD, CW), _f32),
    ),
    mesh=plsc.VectorSubcoreMesh(core_axis_name="c", subcore_axis_name="s",
                                num_cores=NC, num_subcores=NS),
    scratch_types=[
        pltpu.VMEM((C,), jnp.int32),
        pltpu.VMEM((C,), jnp.int32),
        pltpu.VMEM((C, DH), _f32),
        pltpu.VMEM((C, CW), _f32),
        pltpu.VMEM((F, DH), _f32),
        pltpu.VMEM((RPS, CW), _f32),
        pltpu.VMEM_SHARED((N_PAD, DH), _f32),
        pltpu.VMEM_SHARED((N_PAD, CW), _f32),
        pltpu.SemaphoreType.DMA,
    ],
    compiler_params=pltpu.CompilerParams(use_tc_tiling_on_sc=False),
)(_sc_body)


_RB = 1000


def _dense0_body(x_ref, wl_ref, wr_ref, bl_ref, br_ref, ya_ref, yb_ref, r_ref):
    xb = x_ref[...]
    y = jnp.dot(xb, wl_ref[...], preferred_element_type=_f32)
    ya_ref[...] = y[:, :DH]
    yb_ref[...] = y[:, DH:]
    r_ref[...] = (jnp.dot(xb, wr_ref[...], preferred_element_type=_f32)
                  + bl_ref[...] + br_ref[...])


_dense0 = pl.pallas_call(
    _dense0_body,
    grid=(N // _RB,),
    in_specs=[
        pl.BlockSpec((_RB, D), lambda i: (i, 0)),
        pl.BlockSpec((D, D), lambda i: (0, 0)),
        pl.BlockSpec((D, D), lambda i: (0, 0)),
        pl.BlockSpec((1, D), lambda i: (0, 0)),
        pl.BlockSpec((1, D), lambda i: (0, 0)),
    ],
    out_specs=[
        pl.BlockSpec((_RB, DH), lambda i: (i, 0)),
        pl.BlockSpec((_RB, DH), lambda i: (i, 0)),
        pl.BlockSpec((_RB, D), lambda i: (i, 0)),
    ],
    out_shape=[
        jax.ShapeDtypeStruct((N, DH), _f32),
        jax.ShapeDtypeStruct((N, DH), _f32),
        jax.ShapeDtypeStruct((N, D), _f32),
    ],
)


def _mean(sa_ref, sb_ref, c_ref):
    s = jnp.concatenate([sa_ref[...], sb_ref[...]], axis=-1)
    cnt = c_ref[:, 0:1]
    return s / jnp.maximum(cnt, 1.0)


def _combine_mid_body(sa_ref, sb_ref, c_ref, r_ref, wl_ref, wr_ref,
                      bl_ref, br_ref, ya_ref, yb_ref, rn_ref):
    h = jnp.maximum(_mean(sa_ref, sb_ref, c_ref) + r_ref[...], 0.0)
    y = jnp.dot(h, wl_ref[...], preferred_element_type=_f32)
    ya_ref[...] = y[:, :DH]
    yb_ref[...] = y[:, DH:]
    rn_ref[...] = (jnp.dot(h, wr_ref[...], preferred_element_type=_f32)
                   + bl_ref[...] + br_ref[...])


_combine_mid = pl.pallas_call(
    _combine_mid_body,
    grid=(N // _RB,),
    in_specs=[
        pl.BlockSpec((_RB, DH), lambda i: (i, 0)),
        pl.BlockSpec((_RB, DH), lambda i: (i, 0)),
        pl.BlockSpec((_RB, CW), lambda i: (i, 0)),
        pl.BlockSpec((_RB, D), lambda i: (i, 0)),
        pl.BlockSpec((D, D), lambda i: (0, 0)),
        pl.BlockSpec((D, D), lambda i: (0, 0)),
        pl.BlockSpec((1, D), lambda i: (0, 0)),
        pl.BlockSpec((1, D), lambda i: (0, 0)),
    ],
    out_specs=[
        pl.BlockSpec((_RB, DH), lambda i: (i, 0)),
        pl.BlockSpec((_RB, DH), lambda i: (i, 0)),
        pl.BlockSpec((_RB, D), lambda i: (i, 0)),
    ],
    out_shape=[
        jax.ShapeDtypeStruct((N, DH), _f32),
        jax.ShapeDtypeStruct((N, DH), _f32),
        jax.ShapeDtypeStruct((N, D), _f32),
    ],
)


def _combine_out_body(sa_ref, sb_ref, c_ref, r_ref, o_ref):
    z = _mean(sa_ref, sb_ref, c_ref) + r_ref[...]
    m = jnp.max(z, axis=-1, keepdims=True)
    e = jnp.exp(z - m)
    o_ref[...] = (z - m) - jnp.log(jnp.sum(e, axis=-1, keepdims=True))


_combine_out = pl.pallas_call(
    _combine_out_body,
    grid=(N // _RB,),
    in_specs=[
        pl.BlockSpec((_RB, DH), lambda i: (i, 0)),
        pl.BlockSpec((_RB, DH), lambda i: (i, 0)),
        pl.BlockSpec((_RB, CW), lambda i: (i, 0)),
        pl.BlockSpec((_RB, D), lambda i: (i, 0)),
    ],
    out_specs=pl.BlockSpec((_RB, D), lambda i: (i, 0)),
    out_shape=jax.ShapeDtypeStruct((N, D), _f32),
)


def kernel(x, edge_index_l0, edge_index_l1,
           W_l0, b_l0, W_r0, b_r0,
           W_l1, b_l1, W_r1, b_r1):
    src0, dst0 = edge_index_l0[0], edge_index_l0[1]
    src1, dst1 = edge_index_l1[0], edge_index_l1[1]
    bl0 = b_l0.reshape(1, D)
    br0 = b_r0.reshape(1, D)
    bl1 = b_l1.reshape(1, D)
    br1 = b_r1.reshape(1, D)

    ya0, yb0, r0 = _dense0(x, W_l0, W_r0, bl0, br0)
    sa0, sb0, c0 = _sc_agg(ya0, yb0, src0, dst0)
    ya1, yb1, r1 = _combine_mid(sa0, sb0, c0, r0, W_l1, W_r1, bl1, br1)
    sa1, sb1, c1 = _sc_agg(ya1, yb1, src1, dst1)
    return _combine_out(sa1, sb1, c1, r1)


# speedup vs baseline: 10.7752x; 3.2705x over previous
import functools

import jax
import jax.numpy as jnp
from jax import lax
from jax.experimental import pallas as pl
from jax.experimental.pallas import tpu as pltpu
from jax.experimental.pallas import tpu_sc as plsc

N = 10000
E = 320000
D = 128

NC = 2
NS = 16
DH = D // NC
EPS = E // NS
C = 80
NCHUNK = EPS // C
N_PAD = 10240
RPS = N_PAD // NS
F = 128
CW = 16

_f32 = jnp.float32


NB = 5
SEG = 50
NSEG = NCHUNK // SEG


def _sc_body(ya_hbm, yb_hbm, src_hbm, dst_hbm, sa_out, sb_out, cnt_out,
             src_v, dst_v, rows0, rows1, rows2, rows3, rows4,
             ones, zbuf, zcnt, acc_sh, cnt_sh,
             sem0, sem1, sem2, sem3, sem4):
    cid = lax.axis_index("c")
    sid = lax.axis_index("s")
    rows = (rows0, rows1, rows2, rows3, rows4)
    sems = (sem0, sem1, sem2, sem3, sem4)

    def _zrow(i, _):
        for j in range(DH // 16):
            zbuf[i, pl.ds(j * 16, 16)] = jnp.zeros((16,), _f32)
        return 0
    lax.fori_loop(0, F, _zrow, 0)

    def _zcrow(i, _):
        zcnt[i, pl.ds(0, CW)] = jnp.zeros((CW,), _f32)
        return 0
    lax.fori_loop(0, RPS, _zcrow, 0)

    def _orow(i, _):
        ones[i, pl.ds(0, CW)] = jnp.ones((CW,), _f32)
        return 0
    lax.fori_loop(0, C, _orow, 0)

    for k in range(RPS // F):
        pltpu.sync_copy(zbuf, acc_sh.at[pl.ds(sid * RPS + k * F, F)])
    pltpu.sync_copy(zcnt, cnt_sh.at[pl.ds(sid * RPS, RPS)])
    plsc.subcore_barrier()

    def _edges(y_hbm, with_counts):
        def _scatter(b, j):
            pltpu.make_async_copy(y_hbm.at[src_v.at[b]], rows[b],
                                  sems[b]).wait()
            pltpu.sync_copy(rows[b], acc_sh.at[dst_v.at[j]], add=True)
            if with_counts:
                pltpu.sync_copy(ones, cnt_sh.at[dst_v.at[j]], add=True)

        def _segment(s, _):
            pltpu.sync_copy(src_hbm.at[sid, pl.ds(s * SEG, SEG)], src_v)
            pltpu.sync_copy(dst_hbm.at[sid, pl.ds(s * SEG, SEG)], dst_v)
            for b in range(NB):
                pltpu.async_copy(y_hbm.at[src_v.at[b]], rows[b], sems[b])

            def _group(g, _):
                j0 = g * NB
                for b in range(NB):
                    _scatter(b, j0 + b)
                    pltpu.async_copy(y_hbm.at[src_v.at[j0 + b + NB]],
                                     rows[b], sems[b])
                return 0
            lax.fori_loop(0, SEG // NB - 1, _group, 0)
            for b in range(NB):
                _scatter(b, SEG - NB + b)
            return 0
        lax.fori_loop(0, NSEG, _segment, 0)

    @pl.when(cid == 0)
    def _():
        _edges(ya_hbm, True)

    @pl.when(cid == 1)
    def _():
        _edges(yb_hbm, False)

    plsc.subcore_barrier()

    @pl.when(cid == 0)
    def _():
        for k in range(RPS // F):
            r0 = sid * RPS + k * F
            pltpu.sync_copy(acc_sh.at[pl.ds(r0, F)], sa_out.at[pl.ds(r0, F)])
        pltpu.sync_copy(cnt_sh.at[pl.ds(sid * RPS, RPS)],
                        cnt_out.at[pl.ds(sid * RPS, RPS)])

    @pl.when(cid == 1)
    def _():
        for k in range(RPS // F):
            r0 = sid * RPS + k * F
            pltpu.sync_copy(acc_sh.at[pl.ds(r0, F)], sb_out.at[pl.ds(r0, F)])


_sc_agg = functools.partial(
    pl.kernel,
    out_type=(
        jax.ShapeDtypeStruct((N_PAD, DH), _f32),
        jax.ShapeDtypeStruct((N_PAD, DH), _f32),
        jax.ShapeDtypeStruct((N_PAD, CW), _f32),
    ),
    mesh=plsc.VectorSubcoreMesh(core_axis_name="c", subcore_axis_name="s",
                                num_cores=NC, num_subcores=NS),
    scratch_types=[
        pltpu.VMEM((SEG, C), jnp.int32),
        pltpu.VMEM((SEG, C), jnp.int32),
    ] + [pltpu.VMEM((C, DH), _f32) for _ in range(NB)] + [
        pltpu.VMEM((C, CW), _f32),
        pltpu.VMEM((F, DH), _f32),
        pltpu.VMEM((RPS, CW), _f32),
        pltpu.VMEM_SHARED((N_PAD, DH), _f32),
        pltpu.VMEM_SHARED((N_PAD, CW), _f32),
    ] + [pltpu.SemaphoreType.DMA for _ in range(NB)],
    compiler_params=pltpu.CompilerParams(use_tc_tiling_on_sc=False),
)(_sc_body)


_RB = 1000


def _dense0_body(x_ref, wl_ref, wr_ref, bl_ref, br_ref, ya_ref, yb_ref, r_ref):
    xb = x_ref[...]
    y = jnp.dot(xb, wl_ref[...], preferred_element_type=_f32)
    ya_ref[...] = y[:, :DH]
    yb_ref[...] = y[:, DH:]
    r_ref[...] = (jnp.dot(xb, wr_ref[...], preferred_element_type=_f32)
                  + bl_ref[...] + br_ref[...])


_dense0 = pl.pallas_call(
    _dense0_body,
    grid=(N // _RB,),
    in_specs=[
        pl.BlockSpec((_RB, D), lambda i: (i, 0)),
        pl.BlockSpec((D, D), lambda i: (0, 0)),
        pl.BlockSpec((D, D), lambda i: (0, 0)),
        pl.BlockSpec((1, D), lambda i: (0, 0)),
        pl.BlockSpec((1, D), lambda i: (0, 0)),
    ],
    out_specs=[
        pl.BlockSpec((_RB, DH), lambda i: (i, 0)),
        pl.BlockSpec((_RB, DH), lambda i: (i, 0)),
        pl.BlockSpec((_RB, D), lambda i: (i, 0)),
    ],
    out_shape=[
        jax.ShapeDtypeStruct((N, DH), _f32),
        jax.ShapeDtypeStruct((N, DH), _f32),
        jax.ShapeDtypeStruct((N, D), _f32),
    ],
)


def _mean(sa_ref, sb_ref, c_ref):
    s = jnp.concatenate([sa_ref[...], sb_ref[...]], axis=-1)
    cnt = c_ref[:, 0:1]
    return s / jnp.maximum(cnt, 1.0)


def _combine_mid_body(sa_ref, sb_ref, c_ref, r_ref, wl_ref, wr_ref,
                      bl_ref, br_ref, ya_ref, yb_ref, rn_ref):
    h = jnp.maximum(_mean(sa_ref, sb_ref, c_ref) + r_ref[...], 0.0)
    y = jnp.dot(h, wl_ref[...], preferred_element_type=_f32)
    ya_ref[...] = y[:, :DH]
    yb_ref[...] = y[:, DH:]
    rn_ref[...] = (jnp.dot(h, wr_ref[...], preferred_element_type=_f32)
                   + bl_ref[...] + br_ref[...])


_combine_mid = pl.pallas_call(
    _combine_mid_body,
    grid=(N // _RB,),
    in_specs=[
        pl.BlockSpec((_RB, DH), lambda i: (i, 0)),
        pl.BlockSpec((_RB, DH), lambda i: (i, 0)),
        pl.BlockSpec((_RB, CW), lambda i: (i, 0)),
        pl.BlockSpec((_RB, D), lambda i: (i, 0)),
        pl.BlockSpec((D, D), lambda i: (0, 0)),
        pl.BlockSpec((D, D), lambda i: (0, 0)),
        pl.BlockSpec((1, D), lambda i: (0, 0)),
        pl.BlockSpec((1, D), lambda i: (0, 0)),
    ],
    out_specs=[
        pl.BlockSpec((_RB, DH), lambda i: (i, 0)),
        pl.BlockSpec((_RB, DH), lambda i: (i, 0)),
        pl.BlockSpec((_RB, D), lambda i: (i, 0)),
    ],
    out_shape=[
        jax.ShapeDtypeStruct((N, DH), _f32),
        jax.ShapeDtypeStruct((N, DH), _f32),
        jax.ShapeDtypeStruct((N, D), _f32),
    ],
)


def _combine_out_body(sa_ref, sb_ref, c_ref, r_ref, o_ref):
    z = _mean(sa_ref, sb_ref, c_ref) + r_ref[...]
    m = jnp.max(z, axis=-1, keepdims=True)
    e = jnp.exp(z - m)
    o_ref[...] = (z - m) - jnp.log(jnp.sum(e, axis=-1, keepdims=True))


_combine_out = pl.pallas_call(
    _combine_out_body,
    grid=(N // _RB,),
    in_specs=[
        pl.BlockSpec((_RB, DH), lambda i: (i, 0)),
        pl.BlockSpec((_RB, DH), lambda i: (i, 0)),
        pl.BlockSpec((_RB, CW), lambda i: (i, 0)),
        pl.BlockSpec((_RB, D), lambda i: (i, 0)),
    ],
    out_specs=pl.BlockSpec((_RB, D), lambda i: (i, 0)),
    out_shape=jax.ShapeDtypeStruct((N, D), _f32),
)


def kernel(x, edge_index_l0, edge_index_l1,
           W_l0, b_l0, W_r0, b_r0,
           W_l1, b_l1, W_r1, b_r1):
    src0 = edge_index_l0[0].reshape(NS, NCHUNK, C)
    dst0 = edge_index_l0[1].reshape(NS, NCHUNK, C)
    src1 = edge_index_l1[0].reshape(NS, NCHUNK, C)
    dst1 = edge_index_l1[1].reshape(NS, NCHUNK, C)
    bl0 = b_l0.reshape(1, D)
    br0 = b_r0.reshape(1, D)
    bl1 = b_l1.reshape(1, D)
    br1 = b_r1.reshape(1, D)

    ya0, yb0, r0 = _dense0(x, W_l0, W_r0, bl0, br0)
    sa0, sb0, c0 = _sc_agg(ya0, yb0, src0, dst0)
    ya1, yb1, r1 = _combine_mid(sa0, sb0, c0, r0, W_l1, W_r1, bl1, br1)
    sa1, sb1, c1 = _sc_agg(ya1, yb1, src1, dst1)
    return _combine_out(sa1, sb1, c1, r1)


# speedup vs baseline: 10.8618x; 1.0080x over previous
import functools

import jax
import jax.numpy as jnp
from jax import lax
from jax.experimental import pallas as pl
from jax.experimental.pallas import tpu as pltpu
from jax.experimental.pallas import tpu_sc as plsc

N = 10000
E = 320000
D = 128

NC = 2
NS = 16
DH = D // NC
EPS = E // NS
C = 80
NCHUNK = EPS // C
N_PAD = 10240
RPS = N_PAD // NS
F = 128
CW = 16

_f32 = jnp.float32


NB = 5
SEG = 50
NSEG = NCHUNK // SEG


def _sc_body(ya_hbm, yb_hbm, src_hbm, dst_hbm, sa_out, sb_out, cnt_out,
             src_v, dst_v, rows0, rows1, rows2, rows3, rows4,
             ones, zbuf, zcnt, acc_sh, cnt_sh,
             sem0, sem1, sem2, sem3, sem4):
    cid = lax.axis_index("c")
    sid = lax.axis_index("s")
    rows = (rows0, rows1, rows2, rows3, rows4)
    sems = (sem0, sem1, sem2, sem3, sem4)

    def _zrow(i, _):
        for j in range(DH // 16):
            zbuf[i, pl.ds(j * 16, 16)] = jnp.zeros((16,), _f32)
        return 0
    lax.fori_loop(0, F, _zrow, 0)

    def _zcrow(i, _):
        zcnt[i, pl.ds(0, CW)] = jnp.zeros((CW,), _f32)
        return 0
    lax.fori_loop(0, RPS, _zcrow, 0)

    def _orow(i, _):
        ones[i, pl.ds(0, CW)] = jnp.ones((CW,), _f32)
        return 0
    lax.fori_loop(0, C, _orow, 0)

    for k in range(RPS // F):
        pltpu.sync_copy(zbuf, acc_sh.at[pl.ds(sid * RPS + k * F, F)])
    pltpu.sync_copy(zcnt, cnt_sh.at[pl.ds(sid * RPS, RPS)])
    plsc.subcore_barrier()

    def _edges(y_hbm):
        for s in range(NSEG):
            count_core = s % 2

            def _scatter(b, j):
                pltpu.make_async_copy(y_hbm.at[src_v.at[b]], rows[b],
                                      sems[b]).wait()
                pltpu.sync_copy(rows[b], acc_sh.at[dst_v.at[j]], add=True)

                @pl.when(cid == count_core)
                def _():
                    pltpu.sync_copy(ones, cnt_sh.at[dst_v.at[j]], add=True)

            pltpu.sync_copy(src_hbm.at[sid, pl.ds(s * SEG, SEG)], src_v)
            pltpu.sync_copy(dst_hbm.at[sid, pl.ds(s * SEG, SEG)], dst_v)
            for b in range(NB):
                pltpu.async_copy(y_hbm.at[src_v.at[b]], rows[b], sems[b])

            def _group(g, _):
                j0 = g * NB
                for b in range(NB):
                    _scatter(b, j0 + b)
                    pltpu.async_copy(y_hbm.at[src_v.at[j0 + b + NB]],
                                     rows[b], sems[b])
                return 0
            lax.fori_loop(0, SEG // NB - 1, _group, 0)
            for b in range(NB):
                _scatter(b, SEG - NB + b)

    @pl.when(cid == 0)
    def _():
        _edges(ya_hbm)

    @pl.when(cid == 1)
    def _():
        _edges(yb_hbm)

    plsc.subcore_barrier()

    @pl.when(cid == 0)
    def _():
        for k in range(RPS // F):
            r0 = sid * RPS + k * F
            pltpu.sync_copy(acc_sh.at[pl.ds(r0, F)], sa_out.at[pl.ds(r0, F)])

    @pl.when(cid == 1)
    def _():
        for k in range(RPS // F):
            r0 = sid * RPS + k * F
            pltpu.sync_copy(acc_sh.at[pl.ds(r0, F)], sb_out.at[pl.ds(r0, F)])

    pltpu.sync_copy(cnt_sh.at[pl.ds(sid * RPS, RPS)],
                    cnt_out.at[cid, pl.ds(sid * RPS, RPS)])


_sc_agg = functools.partial(
    pl.kernel,
    out_type=(
        jax.ShapeDtypeStruct((N_PAD, DH), _f32),
        jax.ShapeDtypeStruct((N_PAD, DH), _f32),
        jax.ShapeDtypeStruct((NC, N_PAD, CW), _f32),
    ),
    mesh=plsc.VectorSubcoreMesh(core_axis_name="c", subcore_axis_name="s",
                                num_cores=NC, num_subcores=NS),
    scratch_types=[
        pltpu.VMEM((SEG, C), jnp.int32),
        pltpu.VMEM((SEG, C), jnp.int32),
    ] + [pltpu.VMEM((C, DH), _f32) for _ in range(NB)] + [
        pltpu.VMEM((C, CW), _f32),
        pltpu.VMEM((F, DH), _f32),
        pltpu.VMEM((RPS, CW), _f32),
        pltpu.VMEM_SHARED((N_PAD, DH), _f32),
        pltpu.VMEM_SHARED((N_PAD, CW), _f32),
    ] + [pltpu.SemaphoreType.DMA for _ in range(NB)],
    compiler_params=pltpu.CompilerParams(use_tc_tiling_on_sc=False),
)(_sc_body)


_RB = 1000


def _dense0_body(x_ref, wl_ref, wr_ref, bl_ref, br_ref, ya_ref, yb_ref, r_ref):
    xb = x_ref[...]
    y = jnp.dot(xb, wl_ref[...], preferred_element_type=_f32)
    ya_ref[...] = y[:, :DH]
    yb_ref[...] = y[:, DH:]
    r_ref[...] = (jnp.dot(xb, wr_ref[...], preferred_element_type=_f32)
                  + bl_ref[...] + br_ref[...])


_dense0 = pl.pallas_call(
    _dense0_body,
    grid=(N // _RB,),
    in_specs=[
        pl.BlockSpec((_RB, D), lambda i: (i, 0)),
        pl.BlockSpec((D, D), lambda i: (0, 0)),
        pl.BlockSpec((D, D), lambda i: (0, 0)),
        pl.BlockSpec((1, D), lambda i: (0, 0)),
        pl.BlockSpec((1, D), lambda i: (0, 0)),
    ],
    out_specs=[
        pl.BlockSpec((_RB, DH), lambda i: (i, 0)),
        pl.BlockSpec((_RB, DH), lambda i: (i, 0)),
        pl.BlockSpec((_RB, D), lambda i: (i, 0)),
    ],
    out_shape=[
        jax.ShapeDtypeStruct((N, DH), _f32),
        jax.ShapeDtypeStruct((N, DH), _f32),
        jax.ShapeDtypeStruct((N, D), _f32),
    ],
)


def _mean(sa_ref, sb_ref, c_ref):
    s = jnp.concatenate([sa_ref[...], sb_ref[...]], axis=-1)
    cnt = c_ref[0, :, 0:1] + c_ref[1, :, 0:1]
    return s / jnp.maximum(cnt, 1.0)


def _combine_mid_body(sa_ref, sb_ref, c_ref, r_ref, wl_ref, wr_ref,
                      bl_ref, br_ref, ya_ref, yb_ref, rn_ref):
    h = jnp.maximum(_mean(sa_ref, sb_ref, c_ref) + r_ref[...], 0.0)
    y = jnp.dot(h, wl_ref[...], preferred_element_type=_f32)
    ya_ref[...] = y[:, :DH]
    yb_ref[...] = y[:, DH:]
    rn_ref[...] = (jnp.dot(h, wr_ref[...], preferred_element_type=_f32)
                   + bl_ref[...] + br_ref[...])


_combine_mid = pl.pallas_call(
    _combine_mid_body,
    grid=(N // _RB,),
    in_specs=[
        pl.BlockSpec((_RB, DH), lambda i: (i, 0)),
        pl.BlockSpec((_RB, DH), lambda i: (i, 0)),
        pl.BlockSpec((NC, _RB, CW), lambda i: (0, i, 0)),
        pl.BlockSpec((_RB, D), lambda i: (i, 0)),
        pl.BlockSpec((D, D), lambda i: (0, 0)),
        pl.BlockSpec((D, D), lambda i: (0, 0)),
        pl.BlockSpec((1, D), lambda i: (0, 0)),
        pl.BlockSpec((1, D), lambda i: (0, 0)),
    ],
    out_specs=[
        pl.BlockSpec((_RB, DH), lambda i: (i, 0)),
        pl.BlockSpec((_RB, DH), lambda i: (i, 0)),
        pl.BlockSpec((_RB, D), lambda i: (i, 0)),
    ],
    out_shape=[
        jax.ShapeDtypeStruct((N, DH), _f32),
        jax.ShapeDtypeStruct((N, DH), _f32),
        jax.ShapeDtypeStruct((N, D), _f32),
    ],
)


def _combine_out_body(sa_ref, sb_ref, c_ref, r_ref, o_ref):
    z = _mean(sa_ref, sb_ref, c_ref) + r_ref[...]
    m = jnp.max(z, axis=-1, keepdims=True)
    e = jnp.exp(z - m)
    o_ref[...] = (z - m) - jnp.log(jnp.sum(e, axis=-1, keepdims=True))


_combine_out = pl.pallas_call(
    _combine_out_body,
    grid=(N // _RB,),
    in_specs=[
        pl.BlockSpec((_RB, DH), lambda i: (i, 0)),
        pl.BlockSpec((_RB, DH), lambda i: (i, 0)),
        pl.BlockSpec((NC, _RB, CW), lambda i: (0, i, 0)),
        pl.BlockSpec((_RB, D), lambda i: (i, 0)),
    ],
    out_specs=pl.BlockSpec((_RB, D), lambda i: (i, 0)),
    out_shape=jax.ShapeDtypeStruct((N, D), _f32),
)


def kernel(x, edge_index_l0, edge_index_l1,
           W_l0, b_l0, W_r0, b_r0,
           W_l1, b_l1, W_r1, b_r1):
    src0 = edge_index_l0[0].reshape(NS, NCHUNK, C)
    dst0 = edge_index_l0[1].reshape(NS, NCHUNK, C)
    src1 = edge_index_l1[0].reshape(NS, NCHUNK, C)
    dst1 = edge_index_l1[1].reshape(NS, NCHUNK, C)
    bl0 = b_l0.reshape(1, D)
    br0 = b_r0.reshape(1, D)
    bl1 = b_l1.reshape(1, D)
    br1 = b_r1.reshape(1, D)

    ya0, yb0, r0 = _dense0(x, W_l0, W_r0, bl0, br0)
    sa0, sb0, c0 = _sc_agg(ya0, yb0, src0, dst0)
    ya1, yb1, r1 = _combine_mid(sa0, sb0, c0, r0, W_l1, W_r1, bl1, br1)
    sa1, sb1, c1 = _sc_agg(ya1, yb1, src1, dst1)
    return _combine_out(sa1, sb1, c1, r1)
